# baseline (speedup 1.0000x reference)
import functools
import math

import jax
import jax.numpy as jnp
from jax import lax
from jax.experimental import pallas as pl
from jax.experimental.pallas import tpu as pltpu
from jax.experimental.pallas import tpu_sc as plsc

N_BINS = 8192
LOG_N_BINS = math.log(N_BINS)
LN2 = math.log(2.0)

NC = 2
NS = 16
NW = NC * NS
LANES = 16

CHUNK = 16384
NBUF = 2


def _ln_vec(v):
    bits = plsc.bitcast(v, jnp.int32)
    e = lax.shift_right_logical(bits, 23) - 127
    mbits = jnp.bitwise_or(jnp.bitwise_and(bits, 0x007FFFFF), 0x3F800000)
    m = plsc.bitcast(mbits, jnp.float32)
    s = (m - 1.0) / (m + 1.0)
    s2 = s * s
    p = 1.0 + s2 * (1.0 / 3 + s2 * (1.0 / 5 + s2 * (1.0 / 7 + s2 * (1.0 / 9 + s2 / 11))))
    return e.astype(jnp.float32) * LN2 + 2.0 * s * p


def _sc_body(n_samples, x_hbm, logits_hbm, out_hbm, tab_v, x_v, out_v,
             sems_in, sems_out):
    per_w = n_samples // NW
    n_chunks = per_w // CHUNK
    c = lax.axis_index("c")
    s = lax.axis_index("s")
    wid = s * NC + c
    base = wid * per_w

    for b in range(NBUF):
        pltpu.async_copy(
            x_hbm.at[pl.ds(base + b * CHUNK, CHUNK)], x_v.at[b], sems_in.at[b])

    pltpu.sync_copy(logits_hbm, tab_v)
    n_tvec = N_BINS // LANES

    def max_body(i, acc):
        return jnp.maximum(acc, tab_v[pl.ds(i * LANES, LANES)])

    mvec = lax.fori_loop(0, n_tvec, max_body,
                         jnp.full((LANES,), -jnp.inf, jnp.float32))
    mmax = jnp.max(mvec)

    def sum_body(i, acc):
        return acc + jnp.exp(tab_v[pl.ds(i * LANES, LANES)] - mmax)

    svec = lax.fori_loop(0, n_tvec, sum_body, jnp.zeros((LANES,), jnp.float32))
    ssum = jnp.sum(svec)
    shift = (LOG_N_BINS - mmax) - _ln_vec(jnp.full((LANES,), ssum))

    def tab_body(i, _):
        tab_v[pl.ds(i * LANES, LANES)] = tab_v[pl.ds(i * LANES, LANES)] + shift
        return 0

    lax.fori_loop(0, n_tvec, tab_body, 0)

    def ring_body(j, _):
        for b in range(NBUF):
            ch = j * NBUF + b
            off = base + ch * CHUNK
            pltpu.make_async_copy(
                x_hbm.at[pl.ds(base, CHUNK)], x_v.at[b], sems_in.at[b]).wait()
            @pl.when(j > 0)
            def _():
                pltpu.make_async_copy(
                    out_v.at[b], out_hbm.at[pl.ds(base, CHUNK)],
                    sems_out.at[b]).wait()

            @plsc.parallel_loop(0, CHUNK, step=LANES, unroll=16)
            def _(i):
                xv = x_v[b, pl.ds(i, LANES)]
                idx = (xv * float(N_BINS)).astype(jnp.int32)
                idx = jnp.minimum(jnp.maximum(idx, 0), N_BINS - 1)
                out_v[b, pl.ds(i, LANES)] = plsc.load_gather(tab_v, [idx])

            pltpu.async_copy(out_v.at[b], out_hbm.at[pl.ds(off, CHUNK)],
                             sems_out.at[b])
            @pl.when(ch + NBUF < n_chunks)
            def _():
                pltpu.async_copy(
                    x_hbm.at[pl.ds(off + NBUF * CHUNK, CHUNK)], x_v.at[b],
                    sems_in.at[b])
        return 0

    lax.fori_loop(0, n_chunks // NBUF, ring_body, 0)

    for b in range(NBUF):
        pltpu.make_async_copy(
            out_v.at[b], out_hbm.at[pl.ds(base, CHUNK)], sems_out.at[b]).wait()


def kernel(x, logits):
    n = x.shape[0]
    mesh = plsc.VectorSubcoreMesh(core_axis_name="c", subcore_axis_name="s")
    sc = pl.kernel(
        functools.partial(_sc_body, n),
        out_type=jax.ShapeDtypeStruct((n,), jnp.float32),
        mesh=mesh,
        compiler_params=pltpu.CompilerParams(needs_layout_passes=False),
        scratch_types=[
            pltpu.VMEM((N_BINS,), jnp.float32),
            pltpu.VMEM((NBUF, CHUNK), jnp.float32),
            pltpu.VMEM((NBUF, CHUNK), jnp.float32),
            pltpu.SemaphoreType.DMA((NBUF,)),
            pltpu.SemaphoreType.DMA((NBUF,)),
        ],
    )
    return sc(x, logits)


# speedup vs baseline: 1.0310x; 1.0310x over previous
import functools
import math

import jax
import jax.numpy as jnp
from jax import lax
from jax.experimental import pallas as pl
from jax.experimental.pallas import tpu as pltpu
from jax.experimental.pallas import tpu_sc as plsc

N_BINS = 8192
LOG_N_BINS = math.log(N_BINS)
LN2 = math.log(2.0)

NC = 2
NS = 16
NW = NC * NS
LANES = 16

CHUNK = 8192
NBUF = 4


def _ln_vec(v):
    bits = plsc.bitcast(v, jnp.int32)
    e = lax.shift_right_logical(bits, 23) - 127
    mbits = jnp.bitwise_or(jnp.bitwise_and(bits, 0x007FFFFF), 0x3F800000)
    m = plsc.bitcast(mbits, jnp.float32)
    s = (m - 1.0) / (m + 1.0)
    s2 = s * s
    p = 1.0 + s2 * (1.0 / 3 + s2 * (1.0 / 5 + s2 * (1.0 / 7 + s2 * (1.0 / 9 + s2 / 11))))
    return e.astype(jnp.float32) * LN2 + 2.0 * s * p


def _sc_body(n_samples, x_hbm, logits_hbm, out_hbm, tab_v, x_v, out_v,
             sems_in, sems_out):
    per_w = n_samples // NW
    n_chunks = per_w // CHUNK
    c = lax.axis_index("c")
    s = lax.axis_index("s")
    wid = s * NC + c
    base = wid * per_w

    for b in range(NBUF):
        pltpu.async_copy(
            x_hbm.at[pl.ds(base + b * CHUNK, CHUNK)], x_v.at[b], sems_in.at[b])

    pltpu.sync_copy(logits_hbm, tab_v)
    n_tvec = N_BINS // LANES

    @plsc.parallel_loop(0, N_BINS, step=LANES, unroll=8,
                        carry=jnp.full((LANES,), -jnp.inf, jnp.float32))
    def mvec(i, acc):
        return jnp.maximum(acc, tab_v[pl.ds(i, LANES)])

    mmax = jnp.max(mvec)

    @plsc.parallel_loop(0, N_BINS, step=LANES, unroll=8,
                        carry=jnp.zeros((LANES,), jnp.float32))
    def svec(i, acc):
        return acc + jnp.exp(tab_v[pl.ds(i, LANES)] - mmax)

    ssum = jnp.sum(svec)
    shift = (LOG_N_BINS - mmax) - _ln_vec(jnp.full((LANES,), ssum))

    @plsc.parallel_loop(0, N_BINS, step=LANES, unroll=8)
    def _(i):
        tab_v[pl.ds(i, LANES)] = tab_v[pl.ds(i, LANES)] + shift

    def ring_body(j, _):
        for b in range(NBUF):
            ch = j * NBUF + b
            off = base + ch * CHUNK
            pltpu.make_async_copy(
                x_hbm.at[pl.ds(base, CHUNK)], x_v.at[b], sems_in.at[b]).wait()
            @pl.when(j > 0)
            def _():
                pltpu.make_async_copy(
                    out_v.at[b], out_hbm.at[pl.ds(base, CHUNK)],
                    sems_out.at[b]).wait()

            @plsc.parallel_loop(0, CHUNK, step=LANES, unroll=16)
            def _(i):
                xv = x_v[b, pl.ds(i, LANES)]
                idx = (xv * float(N_BINS)).astype(jnp.int32)
                idx = jnp.minimum(jnp.maximum(idx, 0), N_BINS - 1)
                out_v[b, pl.ds(i, LANES)] = plsc.load_gather(tab_v, [idx])

            pltpu.async_copy(out_v.at[b], out_hbm.at[pl.ds(off, CHUNK)],
                             sems_out.at[b])
            @pl.when(ch + NBUF < n_chunks)
            def _():
                pltpu.async_copy(
                    x_hbm.at[pl.ds(off + NBUF * CHUNK, CHUNK)], x_v.at[b],
                    sems_in.at[b])
        return 0

    lax.fori_loop(0, n_chunks // NBUF, ring_body, 0)

    for b in range(NBUF):
        pltpu.make_async_copy(
            out_v.at[b], out_hbm.at[pl.ds(base, CHUNK)], sems_out.at[b]).wait()


def kernel(x, logits):
    n = x.shape[0]
    mesh = plsc.VectorSubcoreMesh(core_axis_name="c", subcore_axis_name="s")
    sc = pl.kernel(
        functools.partial(_sc_body, n),
        out_type=jax.ShapeDtypeStruct((n,), jnp.float32),
        mesh=mesh,
        compiler_params=pltpu.CompilerParams(needs_layout_passes=False),
        scratch_types=[
            pltpu.VMEM((N_BINS,), jnp.float32),
            pltpu.VMEM((NBUF, CHUNK), jnp.float32),
            pltpu.VMEM((NBUF, CHUNK), jnp.float32),
            pltpu.SemaphoreType.DMA((NBUF,)),
            pltpu.SemaphoreType.DMA((NBUF,)),
        ],
    )
    return sc(x, logits)


# speedup vs baseline: 1.0452x; 1.0138x over previous
import functools
import math

import jax
import jax.numpy as jnp
from jax import lax
from jax.experimental import pallas as pl
from jax.experimental.pallas import tpu as pltpu
from jax.experimental.pallas import tpu_sc as plsc

N_BINS = 8192
LOG_N_BINS = math.log(N_BINS)
LN2 = math.log(2.0)

NC = 2
NS = 16
NW = NC * NS
LANES = 16

CHUNK = 16384
NBUF = 2


def _ln_vec(v):
    bits = plsc.bitcast(v, jnp.int32)
    e = lax.shift_right_logical(bits, 23) - 127
    mbits = jnp.bitwise_or(jnp.bitwise_and(bits, 0x007FFFFF), 0x3F800000)
    m = plsc.bitcast(mbits, jnp.float32)
    s = (m - 1.0) / (m + 1.0)
    s2 = s * s
    p = 1.0 + s2 * (1.0 / 3 + s2 * (1.0 / 5 + s2 * (1.0 / 7 + s2 * (1.0 / 9 + s2 / 11))))
    return e.astype(jnp.float32) * LN2 + 2.0 * s * p


def _sc_body(n_samples, x_hbm, logits_hbm, out_hbm, tab_v, x_v, out_v,
             sems_in, sems_out):
    per_w = n_samples // NW
    n_chunks = per_w // CHUNK
    c = lax.axis_index("c")
    s = lax.axis_index("s")
    wid = s * NC + c
    base = wid * per_w

    for b in range(NBUF):
        pltpu.async_copy(
            x_hbm.at[pl.ds(base + b * CHUNK, CHUNK)], x_v.at[b], sems_in.at[b])

    pltpu.sync_copy(logits_hbm, tab_v)
    n_tvec = N_BINS // LANES

    @plsc.parallel_loop(0, N_BINS, step=LANES, unroll=8,
                        carry=jnp.full((LANES,), -jnp.inf, jnp.float32))
    def mvec(i, acc):
        return jnp.maximum(acc, tab_v[pl.ds(i, LANES)])

    mmax = jnp.max(mvec)

    @plsc.parallel_loop(0, N_BINS, step=LANES, unroll=8,
                        carry=jnp.zeros((LANES,), jnp.float32))
    def svec(i, acc):
        return acc + jnp.exp(tab_v[pl.ds(i, LANES)] - mmax)

    ssum = jnp.sum(svec)
    shift = (LOG_N_BINS - mmax) - _ln_vec(jnp.full((LANES,), ssum))

    @plsc.parallel_loop(0, N_BINS, step=LANES, unroll=8)
    def _(i):
        tab_v[pl.ds(i, LANES)] = tab_v[pl.ds(i, LANES)] + shift

    def ring_body(j, _):
        for b in range(NBUF):
            ch = j * NBUF + b
            off = base + ch * CHUNK
            pltpu.make_async_copy(
                x_hbm.at[pl.ds(base, CHUNK)], x_v.at[b], sems_in.at[b]).wait()
            @pl.when(j > 0)
            def _():
                pltpu.make_async_copy(
                    out_v.at[b], out_hbm.at[pl.ds(base, CHUNK)],
                    sems_out.at[b]).wait()

            @plsc.parallel_loop(0, CHUNK, step=LANES, unroll=16)
            def _(i):
                xv = x_v[b, pl.ds(i, LANES)]
                idx = (xv * float(N_BINS)).astype(jnp.int32)
                idx = jnp.minimum(jnp.maximum(idx, 0), N_BINS - 1)
                out_v[b, pl.ds(i, LANES)] = plsc.load_gather(tab_v, [idx])

            pltpu.async_copy(out_v.at[b], out_hbm.at[pl.ds(off, CHUNK)],
                             sems_out.at[b])
            @pl.when(ch + NBUF < n_chunks)
            def _():
                pltpu.async_copy(
                    x_hbm.at[pl.ds(off + NBUF * CHUNK, CHUNK)], x_v.at[b],
                    sems_in.at[b])
        return 0

    lax.fori_loop(0, n_chunks // NBUF, ring_body, 0)

    for b in range(NBUF):
        pltpu.make_async_copy(
            out_v.at[b], out_hbm.at[pl.ds(base, CHUNK)], sems_out.at[b]).wait()


def kernel(x, logits):
    n = x.shape[0]
    mesh = plsc.VectorSubcoreMesh(core_axis_name="c", subcore_axis_name="s")
    sc = pl.kernel(
        functools.partial(_sc_body, n),
        out_type=jax.ShapeDtypeStruct((n,), jnp.float32),
        mesh=mesh,
        compiler_params=pltpu.CompilerParams(needs_layout_passes=False),
        scratch_types=[
            pltpu.VMEM((N_BINS,), jnp.float32),
            pltpu.VMEM((NBUF, CHUNK), jnp.float32),
            pltpu.VMEM((NBUF, CHUNK), jnp.float32),
            pltpu.SemaphoreType.DMA((NBUF,)),
            pltpu.SemaphoreType.DMA((NBUF,)),
        ],
    )
    return sc(x, logits)


# speedup vs baseline: 1.0576x; 1.0119x over previous
import functools
import math

import jax
import jax.numpy as jnp
from jax import lax
from jax.experimental import pallas as pl
from jax.experimental.pallas import tpu as pltpu
from jax.experimental.pallas import tpu_sc as plsc

N_BINS = 8192
LOG_N_BINS = math.log(N_BINS)

NC = 2
NS = 16
NW = NC * NS
LANES = 16

CHUNK = 16384
NBUF_IN = 4
NBUF_OUT = 2


def _table_body(logits_ref, out_ref):
    l = logits_ref[...]
    m = jnp.max(l)
    lse = jnp.log(jnp.sum(jnp.exp(l - m))) + m
    out_ref[...] = l - lse + LOG_N_BINS


def _build_table(logits):
    l2d = logits.reshape(64, 128)
    out = pl.pallas_call(
        _table_body,
        out_shape=jax.ShapeDtypeStruct((64, 128), jnp.float32),
    )(l2d)
    return out.reshape(N_BINS)


def _sc_body(n_samples, x_hbm, tab_hbm, out_hbm, tab_v, x_v, out_v,
             sems_in, sems_out):
    per_w = n_samples // NW
    n_chunks = per_w // CHUNK
    c = lax.axis_index("c")
    s = lax.axis_index("s")
    wid = s * NC + c
    base = wid * per_w

    for b in range(NBUF_IN - 1):
        pltpu.async_copy(
            x_hbm.at[pl.ds(base + b * CHUNK, CHUNK)], x_v.at[b],
            sems_in.at[b])

    pltpu.sync_copy(tab_hbm, tab_v)

    def ring_body(j, _):
        for b in range(NBUF_IN):
            ch = j * NBUF_IN + b
            off = base + ch * CHUNK
            ob = b % NBUF_OUT
            pf = ch + NBUF_IN - 1
            @pl.when(pf < n_chunks)
            def _():
                pltpu.async_copy(
                    x_hbm.at[pl.ds(base + pf * CHUNK, CHUNK)],
                    x_v.at[(b + NBUF_IN - 1) % NBUF_IN],
                    sems_in.at[(b + NBUF_IN - 1) % NBUF_IN])
            pltpu.make_async_copy(
                x_hbm.at[pl.ds(base, CHUNK)], x_v.at[b], sems_in.at[b]).wait()
            @pl.when(ch >= NBUF_OUT)
            def _():
                pltpu.make_async_copy(
                    out_v.at[ob], out_hbm.at[pl.ds(base, CHUNK)],
                    sems_out.at[ob]).wait()

            @plsc.parallel_loop(0, CHUNK, step=LANES, unroll=16)
            def _(i):
                xv = x_v[b, pl.ds(i, LANES)]
                idx = (xv * float(N_BINS)).astype(jnp.int32)
                idx = jnp.minimum(jnp.maximum(idx, 0), N_BINS - 1)
                out_v[ob, pl.ds(i, LANES)] = plsc.load_gather(tab_v, [idx])

            pltpu.async_copy(out_v.at[ob], out_hbm.at[pl.ds(off, CHUNK)],
                             sems_out.at[ob])
        return 0

    lax.fori_loop(0, n_chunks // NBUF_IN, ring_body, 0)

    for ob in range(NBUF_OUT):
        pltpu.make_async_copy(
            out_v.at[ob], out_hbm.at[pl.ds(base, CHUNK)], sems_out.at[ob]).wait()


def kernel(x, logits):
    n = x.shape[0]
    table = _build_table(logits)
    mesh = plsc.VectorSubcoreMesh(core_axis_name="c", subcore_axis_name="s")
    sc = pl.kernel(
        functools.partial(_sc_body, n),
        out_type=jax.ShapeDtypeStruct((n,), jnp.float32),
        mesh=mesh,
        compiler_params=pltpu.CompilerParams(needs_layout_passes=False),
        scratch_types=[
            pltpu.VMEM((N_BINS,), jnp.float32),
            pltpu.VMEM((NBUF_IN, CHUNK), jnp.float32),
            pltpu.VMEM((NBUF_OUT, CHUNK), jnp.float32),
            pltpu.SemaphoreType.DMA((NBUF_IN,)),
            pltpu.SemaphoreType.DMA((NBUF_OUT,)),
        ],
    )
    return sc(x, table)
